# baseline (speedup 1.0000x reference)
import functools

import jax
import jax.numpy as jnp
import numpy as np
from jax import lax
from jax.experimental import pallas as pl
from jax.experimental.pallas import tpu as pltpu
from jax.experimental.pallas import tpu_sc as plsc

B = 16384
F = 26
FH = 13
V = 100000
D = 64
NC = 2
NS = 16
NW = NC * NS
CH = 128
LANES = 16

N_H = B * FH
ROWS_H = N_H // NW
NCH_H = ROWS_H // CH

W = 16384
S = W // 2
NBLK = -(-V // W)
V2 = NBLK * W


def _transpose_block(in_ref, out_ref):
    at = in_ref[0].T
    out_ref[0, :, :D] = at[:S]
    out_ref[0, :, D:] = at[S:]


def _make_transpose(off):
    return pl.pallas_call(
        _transpose_block,
        grid=(FH, NBLK),
        in_specs=[pl.BlockSpec((1, D, W), lambda f, c: (f + off, 0, c))],
        out_specs=pl.BlockSpec((1, S, 128), lambda f, c: (f, c, 0)),
        out_shape=jax.ShapeDtypeStruct((FH, V2 // 2, 128), jnp.float32),
    )


_tp_lo = _make_transpose(0)
_tp_hi = _make_transpose(FH)

GROUP = 2
GROWS = GROUP * CH
NG = NCH_H // GROUP

_mesh = plsc.VectorSubcoreMesh(
    core_axis_name="c", subcore_axis_name="s", num_cores=NC, num_subcores=NS
)


@functools.partial(
    pl.kernel,
    out_type=jax.ShapeDtypeStruct((N_H, D), jnp.float32),
    mesh=_mesh,
    compiler_params=pltpu.CompilerParams(use_tc_tiling_on_sc=False),
    scratch_types=[
        pltpu.VMEM((NCH_H, CH), jnp.int32),
        pltpu.VMEM((NCH_H, CH), jnp.int32),
        pltpu.VMEM((GROWS, D), jnp.float32),
        pltpu.VMEM((GROWS, D), jnp.float32),
        pltpu.SemaphoreType.DMA,
        pltpu.SemaphoreType.DMA,
        pltpu.SemaphoreType.DMA,
        pltpu.SemaphoreType.DMA,
    ],
)
def _gather_kernel(x_hbm, offs_hbm, tab_hbm, out_hbm,
                   idx_v, offs_v, buf0, buf1, gsem0, gsem1, osem0, osem1):
    wid = lax.axis_index("s") * NC + lax.axis_index("c")
    base = wid * ROWS_H

    pltpu.sync_copy(x_hbm.at[wid], idx_v)
    pltpu.sync_copy(offs_hbm, offs_v)

    def add_row(r, carry):
        for j in range(CH // LANES):
            sl = pl.ds(j * LANES, LANES)
            v = idx_v[r, sl]
            w = jnp.bitwise_and(v, W - 1)
            rr = jnp.bitwise_and(w, S - 1)
            jj = lax.shift_right_logical(w, np.int32(S.bit_length() - 1))
            idx_v[r, sl] = offs_v[r, sl] + (v - w) + rr + rr + jj
        return carry

    lax.fori_loop(0, NCH_H, add_row, 0)

    bufs = (buf0, buf1)
    gsems = (gsem0, gsem1)
    osems = (osem0, osem1)

    def fire_group(g, b):
        for j in range(GROUP):
            pltpu.async_copy(
                tab_hbm.at[idx_v.at[g * GROUP + j]],
                bufs[b].at[pl.ds(j * CH, CH)],
                gsems[b],
            )

    def wait_group(b):
        pltpu.make_async_copy(tab_hbm.at[pl.ds(0, GROWS)], bufs[b],
                              gsems[b]).wait()

    def fire_write(g, b):
        pltpu.async_copy(bufs[b], out_hbm.at[pl.ds(base + g * GROWS, GROWS)],
                         osems[b])

    def wait_write(g, b):
        pltpu.make_async_copy(bufs[b],
                              out_hbm.at[pl.ds(base + g * GROWS, GROWS)],
                              osems[b]).wait()

    fire_group(0, 0)

    def pipe(i, carry):
        g0 = 2 * i
        wait_group(0)
        fire_write(g0, 0)

        @pl.when(g0 >= 1)
        def _():
            wait_write(g0 - 1, 1)

        fire_group(g0 + 1, 1)

        g1 = g0 + 1
        wait_group(1)
        fire_write(g1, 1)

        @pl.when(g1 + 1 < NG)
        def _():
            wait_write(g1 - 1, 0)
            fire_group(g1 + 1, 0)

        return carry

    lax.fori_loop(0, NG // 2, pipe, 0)
    wait_write(NG - 2, 0)
    wait_write(NG - 1, 1)


_OFFS = ((np.arange(ROWS_H, dtype=np.int32) % FH) * V2).reshape(NCH_H, CH)


@jax.jit
def kernel(x, tables):
    if x.ndim <= 1:
        x = x[None, :]
    x = x.astype(jnp.int32)
    xw_lo = x[:, :FH].reshape(NW, NCH_H, CH)
    xw_hi = x[:, FH:].reshape(NW, NCH_H, CH)
    offs = jnp.asarray(_OFFS)
    tables_t = jnp.swapaxes(tables, 1, 2)
    tab_lo = _tp_lo(tables_t).reshape(FH * V2, D)
    out_lo = _gather_kernel(xw_lo, offs, tab_lo)
    tab_hi = _tp_hi(tables_t).reshape(FH * V2, D)
    out_hi = _gather_kernel(xw_hi, offs, tab_hi)
    return jnp.concatenate(
        [out_lo.reshape(B, FH * D), out_hi.reshape(B, FH * D)], axis=1)


# speedup vs baseline: 1.0734x; 1.0734x over previous
import functools

import jax
import jax.numpy as jnp
import numpy as np
from jax import lax
from jax.experimental import pallas as pl
from jax.experimental.pallas import tpu as pltpu
from jax.experimental.pallas import tpu_sc as plsc

B = 16384
F = 26
V = 100000
D = 64
N = B * F
NC = 2
NS = 16
NW = NC * NS
ROWS = N // NW
CH = 128
NCH = ROWS // CH
LANES = 16

W = 16384
S = W // 2
NBLK = -(-V // W)
V2 = NBLK * W


def _transpose_block(in_ref, out_ref):
    at0 = in_ref[0, :, :S].T
    at1 = in_ref[0, :, S:].T
    out_ref[0, :, :D] = at0
    out_ref[0, :, D:] = at1


_tc_transpose = pl.pallas_call(
    _transpose_block,
    grid=(F, NBLK),
    in_specs=[pl.BlockSpec((1, D, W), lambda f, c: (f, 0, c))],
    out_specs=pl.BlockSpec((1, S, 128), lambda f, c: (f, c, 0)),
    out_shape=jax.ShapeDtypeStruct((F, V2 // 2, 128), jnp.float32),
)

GROUP = 4
GROWS = GROUP * CH
NG = ROWS // GROWS

_mesh = plsc.VectorSubcoreMesh(
    core_axis_name="c", subcore_axis_name="s", num_cores=NC, num_subcores=NS
)


@functools.partial(
    pl.kernel,
    out_type=jax.ShapeDtypeStruct((N, D), jnp.float32),
    mesh=_mesh,
    compiler_params=pltpu.CompilerParams(use_tc_tiling_on_sc=False),
    scratch_types=[
        pltpu.VMEM((NCH, CH), jnp.int32),
        pltpu.VMEM((NCH, CH), jnp.int32),
        pltpu.VMEM((GROWS, D), jnp.float32),
        pltpu.VMEM((GROWS, D), jnp.float32),
        pltpu.SemaphoreType.DMA,
        pltpu.SemaphoreType.DMA,
        pltpu.SemaphoreType.DMA,
        pltpu.SemaphoreType.DMA,
    ],
)
def _gather_kernel(x_hbm, offs_hbm, tab_hbm, out_hbm,
                   idx_v, offs_v, buf0, buf1, gsem0, gsem1, osem0, osem1):
    wid = lax.axis_index("s") * NC + lax.axis_index("c")
    base = wid * ROWS

    pltpu.sync_copy(x_hbm.at[wid], idx_v)
    pltpu.sync_copy(offs_hbm, offs_v)

    def add_row(r, carry):
        for j in range(CH // LANES):
            sl = pl.ds(j * LANES, LANES)
            v = idx_v[r, sl]
            w = jnp.bitwise_and(v, W - 1)
            rr = jnp.bitwise_and(w, S - 1)
            jj = lax.shift_right_logical(w, np.int32(S.bit_length() - 1))
            idx_v[r, sl] = offs_v[r, sl] + (v - w) + rr + rr + jj
        return carry

    lax.fori_loop(0, NCH, add_row, 0)

    bufs = (buf0, buf1)
    gsems = (gsem0, gsem1)
    osems = (osem0, osem1)

    def fire_group(g, b):
        for j in range(GROUP):
            pltpu.async_copy(
                tab_hbm.at[idx_v.at[g * GROUP + j]],
                bufs[b].at[pl.ds(j * CH, CH)],
                gsems[b],
            )

    def wait_group(b):
        pltpu.make_async_copy(tab_hbm.at[pl.ds(0, GROWS)], bufs[b],
                              gsems[b]).wait()

    def fire_write(g, b):
        pltpu.async_copy(bufs[b], out_hbm.at[pl.ds(base + g * GROWS, GROWS)],
                         osems[b])

    def wait_write(g, b):
        pltpu.make_async_copy(bufs[b],
                              out_hbm.at[pl.ds(base + g * GROWS, GROWS)],
                              osems[b]).wait()

    fire_group(0, 0)

    def pipe(i, carry):
        g0 = 2 * i
        wait_group(0)
        fire_write(g0, 0)

        @pl.when(g0 >= 1)
        def _():
            wait_write(g0 - 1, 1)

        fire_group(g0 + 1, 1)

        g1 = g0 + 1
        wait_group(1)
        fire_write(g1, 1)

        @pl.when(g1 + 1 < NG)
        def _():
            wait_write(g1 - 1, 0)
            fire_group(g1 + 1, 0)

        return carry

    lax.fori_loop(0, NG // 2, pipe, 0)
    wait_write(NG - 2, 0)
    wait_write(NG - 1, 1)


_OFFS = ((np.arange(ROWS, dtype=np.int32) % F) * V2).reshape(NCH, CH)


@jax.jit
def kernel(x, tables):
    if x.ndim <= 1:
        x = x[None, :]
    xw = x.astype(jnp.int32).reshape(NW, NCH, CH)
    tables_t = jnp.swapaxes(tables, 1, 2)
    tab = _tc_transpose(tables_t).reshape(F * V2, D)
    out = _gather_kernel(xw, jnp.asarray(_OFFS), tab)
    return out.reshape(B, F * D)
